# baseline (speedup 1.0000x reference)
import functools

import jax
import jax.numpy as jnp
from jax import lax
from jax.experimental import pallas as pl
from jax.experimental.pallas import tpu as pltpu
from jax.experimental.pallas import tpu_sc as plsc

VOCAB = 1000000
EMB = 64
B = 4096
S = 200
NC = 2
NS = 16
NW = NC * NS
BT = B // NW
NBT = B // 128
NB = 4
NG = S // NB
SCALE = 1.0 / 64.0
L = 16


@functools.partial(
    pl.kernel,
    mesh=plsc.VectorSubcoreMesh(core_axis_name="c", subcore_axis_name="s"),
    out_type=jax.ShapeDtypeStruct((S, EMB // 8, NBT, 8, 128), jnp.float32),
    compiler_params=pltpu.CompilerParams(
        use_tc_tiling_on_sc=False, needs_layout_passes=False),
    scratch_types=(
        [pltpu.VMEM((S, BT), jnp.int32)]
        + [pltpu.VMEM((BT, EMB), jnp.float32) for _ in range(NB)]
        + [pltpu.VMEM((EMB // 8, 8, BT), jnp.float32) for _ in range(NB)]
        + [pltpu.SemaphoreType.DMA for _ in range(2 * NB)]
    ),
)
def _emb_lookup(idx_hbm, table_hbm, out_hbm, idxv,
                b0, b1, b2, b3, t0, t1, t2, t3,
                g0, g1, g2, g3, o0, o1, o2, o3):
    bufs = (b0, b1, b2, b3)
    ots = (t0, t1, t2, t3)
    gsems = (g0, g1, g2, g3)
    osems = (o0, o1, o2, o3)
    wid = lax.axis_index("s") * NC + lax.axis_index("c")
    boff = wid * BT
    lanes = lax.iota(jnp.int32, L)
    zrow = lanes * 0
    ehi = [lax.shift_right_logical(t * L + lanes, jnp.int32(3))
           for t in range(EMB // L)]
    elo = [lax.bitwise_and(t * L + lanes, jnp.int32(7))
           for t in range(EMB // L)]

    pltpu.sync_copy(idx_hbm.at[pl.ds(0, S), pl.ds(boff, BT)], idxv)

    def tile_dst(s, e8):
        return out_hbm.at[s, e8, wid, pl.ds(0, 8), pl.ds(0, 128)]

    def store_tiles(s, k):
        for e8 in range(EMB // 8):
            pltpu.async_copy(ots[k].at[e8], tile_dst(s, e8), osems[k])

    def wait_store(k):
        for e8 in range(EMB // 8):
            pltpu.make_async_copy(ots[k].at[e8], tile_dst(0, e8),
                                  osems[k]).wait()

    def start_gather(s, k):
        pltpu.async_copy(table_hbm.at[idxv.at[s]], bufs[k], gsems[k])

    def wait_gather(k):
        pltpu.make_async_copy(table_hbm.at[idxv.at[0]], bufs[k], gsems[k]).wait()

    for k in range(NB):
        start_gather(k, k)

    def group_body(g, carry):
        for k in range(NB):
            s = g * NB + k
            with jax.named_scope("wait_gather"):
                wait_gather(k)

            def row_body(b2, c, k=k):
                b = b2 * 2
                bss = (zrow + b, zrow + (b + 1))
                loads = [bufs[k][b + r, pl.ds(t * L, L)]
                         for r in range(2) for t in range(EMB // L)]
                scaled = [v * SCALE for v in loads]
                for r in range(2):
                    for t in range(EMB // L):
                        plsc.store_scatter(
                            ots[k], [ehi[t], elo[t], bss[r]],
                            scaled[r * (EMB // L) + t])
                return c

            with jax.named_scope("transpose_scale"):
                lax.fori_loop(0, BT // 2, row_body, 0)
            with jax.named_scope("store_tiles"):
                store_tiles(s, k)

        @pl.when(g + 1 < NG)
        def _prefetch():
            with jax.named_scope("prefetch"):
                for k in range(NB):
                    wait_store(k)
                    start_gather((g + 1) * NB + k, k)

        return carry

    lax.fori_loop(0, NG, group_body, 0)
    for k in range(NB):
        wait_store(k)


def kernel(inp, table):
    idx_t = jnp.swapaxes(inp, 0, 1)
    out = _emb_lookup(idx_t, table)
    return jnp.transpose(out, (2, 4, 0, 1, 3)).reshape(B, S, EMB)


# speedup vs baseline: 1.6890x; 1.6890x over previous
import functools

import jax
import jax.numpy as jnp
from jax import lax
from jax.experimental import pallas as pl
from jax.experimental.pallas import tpu as pltpu
from jax.experimental.pallas import tpu_sc as plsc

VOCAB = 1000000
EMB = 64
B = 4096
S = 200
NC = 2
NS = 16
NW = NC * NS
BT = B // NW
NBT = B // 128
NB = 4
NG = S // NB
SCALE = 1.0 / 64.0
L = 16


@functools.partial(
    pl.kernel,
    mesh=plsc.VectorSubcoreMesh(core_axis_name="c", subcore_axis_name="s"),
    out_type=jax.ShapeDtypeStruct((S, EMB // 8, NBT, 8, 128), jnp.float32),
    compiler_params=pltpu.CompilerParams(
        use_tc_tiling_on_sc=False, needs_layout_passes=False),
    scratch_types=(
        [pltpu.VMEM((S, BT), jnp.int32)]
        + [pltpu.VMEM((BT, EMB), jnp.float32) for _ in range(NB)]
        + [pltpu.VMEM((EMB // 8, 8, BT + 1), jnp.float32) for _ in range(NB)]
        + [pltpu.SemaphoreType.DMA for _ in range(2 * NB)]
    ),
)
def _emb_lookup(idx_hbm, table_hbm, out_hbm, idxv,
                b0, b1, b2, b3, t0, t1, t2, t3,
                g0, g1, g2, g3, o0, o1, o2, o3):
    bufs = (b0, b1, b2, b3)
    ots = (t0, t1, t2, t3)
    gsems = (g0, g1, g2, g3)
    osems = (o0, o1, o2, o3)
    wid = lax.axis_index("s") * NC + lax.axis_index("c")
    boff = wid * BT
    lanes = lax.iota(jnp.int32, L)
    zrow = lanes * 0
    ehi = [lax.shift_right_logical(t * L + lanes, jnp.int32(3))
           for t in range(EMB // L)]
    elo = [lax.bitwise_and(t * L + lanes, jnp.int32(7))
           for t in range(EMB // L)]

    pltpu.sync_copy(idx_hbm.at[pl.ds(0, S), pl.ds(boff, BT)], idxv)

    def tile_dst(s, e8):
        return out_hbm.at[s, e8, wid, pl.ds(0, 8), pl.ds(0, 128)]

    def store_tiles(s, k):
        for e8 in range(EMB // 8):
            pltpu.async_copy(ots[k].at[e8, pl.ds(0, 8), pl.ds(0, BT)],
                             tile_dst(s, e8), osems[k])

    def wait_store(k):
        for e8 in range(EMB // 8):
            pltpu.make_async_copy(ots[k].at[e8, pl.ds(0, 8), pl.ds(0, BT)],
                                  tile_dst(0, e8), osems[k]).wait()

    def start_gather(s, k):
        pltpu.async_copy(table_hbm.at[idxv.at[s]], bufs[k], gsems[k])

    def wait_gather(k):
        pltpu.make_async_copy(table_hbm.at[idxv.at[0]], bufs[k], gsems[k]).wait()

    for k in range(NB):
        start_gather(k, k)

    def group_body(g, carry):
        for k in range(NB):
            s = g * NB + k
            with jax.named_scope("wait_gather"):
                wait_gather(k)

            def row_body(b2, c, k=k):
                b = b2 * 2
                bss = (zrow + b, zrow + (b + 1))
                loads = [bufs[k][b + r, pl.ds(t * L, L)]
                         for r in range(2) for t in range(EMB // L)]
                scaled = [v * SCALE for v in loads]
                for r in range(2):
                    for t in range(EMB // L):
                        plsc.store_scatter(
                            ots[k], [ehi[t], elo[t], bss[r]],
                            scaled[r * (EMB // L) + t])
                return c

            with jax.named_scope("transpose_scale"):
                lax.fori_loop(0, BT // 2, row_body, 0)
            with jax.named_scope("store_tiles"):
                store_tiles(s, k)

        @pl.when(g + 1 < NG)
        def _prefetch():
            with jax.named_scope("prefetch"):
                for k in range(NB):
                    wait_store(k)
                    start_gather((g + 1) * NB + k, k)

        return carry

    lax.fori_loop(0, NG, group_body, 0)
    for k in range(NB):
        wait_store(k)


def kernel(inp, table):
    idx_t = jnp.swapaxes(inp, 0, 1)
    out = _emb_lookup(idx_t, table)
    return jnp.transpose(out, (2, 4, 0, 1, 3)).reshape(B, S, EMB)


# speedup vs baseline: 1.6971x; 1.0048x over previous
import functools

import jax
import jax.numpy as jnp
from jax import lax
from jax.experimental import pallas as pl
from jax.experimental.pallas import tpu as pltpu
from jax.experimental.pallas import tpu_sc as plsc

VOCAB = 1000000
EMB = 64
B = 4096
S = 200
NC = 2
NS = 16
NW = NC * NS
BT = B // NW
NBT = B // 128
NB = 4
NG = S // NB
SCALE = 1.0 / 64.0
L = 16


@functools.partial(
    pl.kernel,
    mesh=plsc.VectorSubcoreMesh(core_axis_name="c", subcore_axis_name="s"),
    out_type=jax.ShapeDtypeStruct((S, EMB // 8, NBT, 8, 128), jnp.float32),
    compiler_params=pltpu.CompilerParams(
        use_tc_tiling_on_sc=False, needs_layout_passes=False),
    scratch_types=(
        [pltpu.VMEM((S, BT), jnp.int32)]
        + [pltpu.VMEM((BT, EMB), jnp.float32) for _ in range(NB)]
        + [pltpu.VMEM((EMB // 8, 8, BT + 1), jnp.float32) for _ in range(NB)]
        + [pltpu.SemaphoreType.DMA for _ in range(2 * NB)]
    ),
)
def _emb_lookup(idx_hbm, table_hbm, out_hbm, idxv,
                b0, b1, b2, b3, t0, t1, t2, t3,
                g0, g1, g2, g3, o0, o1, o2, o3):
    bufs = (b0, b1, b2, b3)
    ots = (t0, t1, t2, t3)
    gsems = (g0, g1, g2, g3)
    osems = (o0, o1, o2, o3)
    wid = lax.axis_index("s") * NC + lax.axis_index("c")
    boff = wid * BT
    lanes = lax.iota(jnp.int32, L)
    zrow = lanes * 0
    ehi = [lax.shift_right_logical(t * L + lanes, jnp.int32(3))
           for t in range(EMB // L)]
    elo = [lax.bitwise_and(t * L + lanes, jnp.int32(7))
           for t in range(EMB // L)]

    pltpu.sync_copy(idx_hbm.at[pl.ds(0, S), pl.ds(boff, BT)], idxv)

    def tile_dst(s, e8):
        return out_hbm.at[s, e8, wid, pl.ds(0, 8), pl.ds(0, 128)]

    def store_tiles(s, k):
        for e8 in range(EMB // 8):
            pltpu.async_copy(ots[k].at[e8, pl.ds(0, 8), pl.ds(0, BT)],
                             tile_dst(s, e8), osems[k])

    def wait_store(k):
        for e8 in range(EMB // 8):
            pltpu.make_async_copy(ots[k].at[e8, pl.ds(0, 8), pl.ds(0, BT)],
                                  tile_dst(0, e8), osems[k]).wait()

    def start_gather(s, k):
        pltpu.async_copy(table_hbm.at[idxv.at[s]], bufs[k], gsems[k])

    def wait_gather(k):
        pltpu.make_async_copy(table_hbm.at[idxv.at[0]], bufs[k], gsems[k]).wait()

    for k in range(NB):
        start_gather(k, k)

    def group_body(g, carry):
        for k in range(NB):
            s = g * NB + k
            wait_gather(k)

            def row_body(b4, c, k=k):
                b = b4 * 4
                bss = [zrow + (b + r) for r in range(4)]
                loads = [bufs[k][b + r, pl.ds(t * L, L)]
                         for r in range(4) for t in range(EMB // L)]
                scaled = [v * SCALE for v in loads]
                for r in range(4):
                    for t in range(EMB // L):
                        plsc.store_scatter(
                            ots[k], [ehi[t], elo[t], bss[r]],
                            scaled[r * (EMB // L) + t])
                return c

            lax.fori_loop(0, BT // 4, row_body, 0)
            store_tiles(s, k)

        @pl.when(g + 1 < NG)
        def _prefetch():
            for k in range(NB):
                wait_store(k)
                start_gather((g + 1) * NB + k, k)

        return carry

    lax.fori_loop(0, NG, group_body, 0)
    for k in range(NB):
        wait_store(k)


def kernel(inp, table):
    idx_t = jnp.swapaxes(inp, 0, 1)
    out = _emb_lookup(idx_t, table)
    return jnp.transpose(out, (2, 4, 0, 1, 3)).reshape(B, S, EMB)
